# baseline (speedup 1.0000x reference)
import functools

import jax
import jax.numpy as jnp
from jax import lax
from jax.experimental import pallas as pl
from jax.experimental.pallas import tpu as pltpu
from jax.experimental.pallas import tpu_sc as plsc

F32 = jnp.float32
HD = 128
SB = 256
CH = 256
TN = 1000
SC_CHUNK = 512
SC_NW = 32
SC_ALIGN = SC_NW * SC_CHUNK
EPS = 1e-5

N0, N1, N2 = 10000, 160000, 50000
NB = 64


def _ceil_to(x, m):
    return (x + m - 1) // m * m


@functools.lru_cache(maxsize=None)
def _sc_gather(e_pad):
    b_per_w = e_pad // SC_NW
    n_chunks = b_per_w // SC_CHUNK
    mesh = plsc.VectorSubcoreMesh(core_axis_name="c", subcore_axis_name="s")

    @functools.partial(
        pl.kernel,
        mesh=mesh,
        out_type=jax.ShapeDtypeStruct((e_pad, HD), F32),
        scratch_types=[
            pltpu.VMEM((SC_CHUNK,), jnp.int32),
            pltpu.VMEM((SC_CHUNK, HD), F32),
            pltpu.SemaphoreType.DMA,
        ],
    )
    def gather_kernel(table_hbm, idx_hbm, out_hbm, idx_v, rows_v, sem):
        wid = lax.axis_index("s") * 2 + lax.axis_index("c")
        base0 = wid * b_per_w

        def body(j, carry):
            base = base0 + j * SC_CHUNK
            pltpu.sync_copy(idx_hbm.at[pl.ds(base, SC_CHUNK)], idx_v)
            pltpu.async_copy(table_hbm.at[idx_v], rows_v, sem).wait()
            pltpu.sync_copy(rows_v, out_hbm.at[pl.ds(base, SC_CHUNK)])
            return carry

        lax.fori_loop(0, n_chunks, body, 0)

    return gather_kernel


def _segsum_body(starts_ref, ids_hbm, rows_hbm, out_ref, ibuf, gbuf, sem_i,
                 sem_g, *, sb):
    g = pl.program_id(0)
    e0 = starts_ref[g]
    e1 = starts_ref[g + 1]
    c0 = e0 // CH
    c1 = (e1 + CH - 1) // CH
    out_ref[...] = jnp.zeros_like(out_ref)
    row_iota = lax.broadcasted_iota(jnp.int32, (sb, CH), 0)
    base_id = g * sb

    def body(c, carry):
        ci = pltpu.make_async_copy(ids_hbm.at[c], ibuf, sem_i)
        cg = pltpu.make_async_copy(rows_hbm.at[pl.ds(c * CH, CH)], gbuf, sem_g)
        ci.start()
        cg.start()
        ci.wait()
        cg.wait()
        local = ibuf[0:1, :] - base_id
        oh = (row_iota == local).astype(F32)
        out_ref[...] += jnp.dot(oh, gbuf[...], preferred_element_type=F32)
        return carry

    lax.fori_loop(c0, c1, body, 0)


@functools.lru_cache(maxsize=None)
def _segsum(nb, sb, e_pad, interpret=False):
    grid_spec = pltpu.PrefetchScalarGridSpec(
        num_scalar_prefetch=1,
        grid=(nb,),
        in_specs=[
            pl.BlockSpec(memory_space=pl.ANY),
            pl.BlockSpec(memory_space=pl.ANY),
        ],
        out_specs=pl.BlockSpec((sb, HD), lambda g, starts: (g, 0)),
        scratch_shapes=[
            pltpu.VMEM((1, CH), jnp.int32),
            pltpu.VMEM((CH, HD), F32),
            pltpu.SemaphoreType.DMA,
            pltpu.SemaphoreType.DMA,
        ],
    )
    return pl.pallas_call(
        functools.partial(_segsum_body, sb=sb),
        grid_spec=grid_spec,
        out_shape=jax.ShapeDtypeStruct((nb * sb, HD), F32),
        interpret=interpret,
    )


def _affrelu(y, stats_ref, gb_ref, n_rows):
    inv_n = jnp.float32(1.0 / n_rows)
    mean = stats_ref[0:1, :] * inv_n
    var = stats_ref[1:2, :] * inv_n - mean * mean
    scale = gb_ref[0:1, :] * lax.rsqrt(var + EPS)
    shift = gb_ref[1:2, :] - mean * scale
    return jnp.maximum(y * scale + shift, 0.0)


def _acc_stats(y, st_ref):
    @pl.when(pl.program_id(0) == 0)
    def _init():
        st_ref[...] = jnp.zeros_like(st_ref)

    st_ref[0:1, :] += jnp.sum(y, axis=0, keepdims=True)
    st_ref[1:2, :] += jnp.sum(y * y, axis=0, keepdims=True)


def _s1_add_body(a_ref, b_ref, w_ref, bias_ref, y_ref, st_ref):
    x = a_ref[...] + b_ref[...]
    y = jnp.dot(x, w_ref[...], preferred_element_type=F32) + bias_ref[0:1, :]
    y_ref[...] = y
    _acc_stats(y, st_ref)


def _s1_single_body(a_ref, w_ref, bias_ref, y_ref, st_ref):
    y = (jnp.dot(a_ref[...], w_ref[...], preferred_element_type=F32)
         + bias_ref[0:1, :])
    y_ref[...] = y
    _acc_stats(y, st_ref)


def _s2_body(a_ref, sta_ref, gba_ref, w_ref, bias_ref, y_ref, st_ref, *,
             n_rows):
    x = _affrelu(a_ref[...], sta_ref, gba_ref, n_rows)
    y = jnp.dot(x, w_ref[...], preferred_element_type=F32) + bias_ref[0:1, :]
    y_ref[...] = y
    _acc_stats(y, st_ref)


def _s3_body(a_ref, sta_ref, gba_ref, b_ref, stb_ref, gbb_ref, w1_ref, w2_ref,
             bias_ref, y_ref, st_ref, *, n_rows):
    xa = _affrelu(a_ref[...], sta_ref, gba_ref, n_rows)
    xb = _affrelu(b_ref[...], stb_ref, gbb_ref, n_rows)
    y = (jnp.dot(xa, w1_ref[...], preferred_element_type=F32)
         + jnp.dot(xb, w2_ref[...], preferred_element_type=F32)
         + bias_ref[0:1, :])
    y_ref[...] = y
    _acc_stats(y, st_ref)


def _fin_body(a_ref, sta_ref, gba_ref, y_ref, *, n_rows):
    y_ref[...] = _affrelu(a_ref[...], sta_ref, gba_ref, n_rows)


def _row_spec():
    return pl.BlockSpec((TN, HD), lambda i: (i, 0))


def _full_spec(shape):
    return pl.BlockSpec(shape, lambda i: tuple(0 for _ in shape))


def _stats_out_specs():
    return [
        pl.BlockSpec((TN, HD), lambda i: (i, 0)),
        pl.BlockSpec((8, HD), lambda i: (0, 0)),
    ]


def _stats_out_shapes():
    return [
        None,
        jax.ShapeDtypeStruct((8, HD), F32),
    ]


@functools.lru_cache(maxsize=None)
def _stage(mode, n_rows, interpret=False):
    grid = (n_rows // TN,)
    y_shape = jax.ShapeDtypeStruct((n_rows, HD), F32)
    st_shape = jax.ShapeDtypeStruct((8, HD), F32)
    small = _full_spec((8, HD))
    wspec = _full_spec((HD, HD))
    if mode == "s1_add":
        body = _s1_add_body
        in_specs = [_row_spec(), _row_spec(), wspec, small]
    elif mode == "s1_single":
        body = _s1_single_body
        in_specs = [_row_spec(), wspec, small]
    elif mode == "s2":
        body = functools.partial(_s2_body, n_rows=n_rows)
        in_specs = [_row_spec(), small, small, wspec, small]
    elif mode == "s3":
        body = functools.partial(_s3_body, n_rows=n_rows)
        in_specs = [_row_spec(), small, small, _row_spec(), small, small,
                    wspec, wspec, small]
    elif mode == "fin":
        body = functools.partial(_fin_body, n_rows=n_rows)
        in_specs = [_row_spec(), small, small]
        return pl.pallas_call(
            body,
            grid=grid,
            in_specs=in_specs,
            out_specs=_row_spec(),
            out_shape=y_shape,
            interpret=interpret,
        )
    else:
        raise ValueError(mode)
    return pl.pallas_call(
        body,
        grid=grid,
        in_specs=in_specs,
        out_specs=[_row_spec(), pl.BlockSpec((8, HD), lambda i: (0, 0))],
        out_shape=[y_shape, st_shape],
        interpret=interpret,
    )


def _readout_body(p0, p1, p2, w10, b10, w11, b11, w12, b12, w2, b2, out_ref):
    h = jnp.maximum(jnp.dot(p0[...], w10[...],
                            preferred_element_type=F32) + b10[0:1, :], 0.0)
    h += jnp.maximum(jnp.dot(p1[...], w11[...],
                             preferred_element_type=F32) + b11[0:1, :], 0.0)
    h += jnp.maximum(jnp.dot(p2[...], w12[...],
                             preferred_element_type=F32) + b12[0:1, :], 0.0)
    out_ref[...] = (jnp.dot(h, w2[...], preferred_element_type=F32)
                    + b2[0:1, :])


@functools.lru_cache(maxsize=None)
def _readout(interpret=False):
    return pl.pallas_call(
        _readout_body,
        out_shape=jax.ShapeDtypeStruct((NB, HD), F32),
        interpret=interpret,
    )


def _pack2(top, bot):
    z = jnp.zeros((8, top.shape[0]), F32)
    return z.at[0].set(top).at[1].set(bot)


def _pack1(v):
    return jnp.zeros((8, v.shape[0]), F32).at[0].set(v)


def _prep_edges(src, dst, n_out):
    e = src.shape[0]
    nb = (n_out + SB - 1) // SB
    order = jnp.argsort(dst)
    src_s = jnp.take(src, order).astype(jnp.int32)
    dst_s = jnp.take(dst, order).astype(jnp.int32)
    e_pad = _ceil_to(e, SC_ALIGN)
    sent = jnp.int32(nb * SB)
    src_p = jnp.concatenate(
        [src_s, jnp.zeros((e_pad - e,), jnp.int32)])
    ids_p = jnp.concatenate(
        [dst_s, jnp.full((e_pad - e,), sent, jnp.int32)])
    ids3 = ids_p.reshape(e_pad // CH, 1, CH)
    bounds = jnp.arange(nb + 1, dtype=jnp.int32) * SB
    starts = jnp.searchsorted(dst_s, bounds).astype(jnp.int32)
    return (src_p, ids3, starts, nb, e_pad)


def _msg(table, prep, n_out):
    src_p, ids3, starts, nb, e_pad = prep
    rows = _sc_gather(e_pad)(table, src_p)
    out = _segsum(nb, SB, e_pad)(starts, ids3, rows)
    return out[:n_out]


def _prep_pool(batch, n):
    e_pad = _ceil_to(n, CH)
    ids_p = jnp.concatenate(
        [batch.astype(jnp.int32),
         jnp.full((e_pad - n,), NB, jnp.int32)])
    ids3 = ids_p.reshape(e_pad // CH, 1, CH)
    starts = jnp.array([0, n], jnp.int32)
    return ids3, starts, e_pad


def _pool(x, prep):
    ids3, starts, e_pad = prep
    xp = jnp.pad(x, ((0, e_pad - x.shape[0]), (0, 0)))
    return _segsum(1, NB, e_pad)(starts, ids3, xp)


def _mlp_pair(x, msg, p, n):
    w1 = p['W1']
    if msg is None:
        y1, st1 = _stage("s1_single", n)(x, w1, _pack1(p['b1']))
    else:
        y1, st1 = _stage("s1_add", n)(x, msg, w1, _pack1(p['b1']))
    y2, st2 = _stage("s2", n)(y1, st1, _pack2(p['g1'], p['be1']), p['W2'],
                              _pack1(p['b2']))
    return y2, st2, _pack2(p['g2'], p['be2'])


def _conv_dim(x, up_msg, b_msg, p, n):
    yu, stu, gbu = _mlp_pair(x, up_msg, p['up'], n)
    yb, stb, gbb = _mlp_pair(x, b_msg, p['bnd'], n)
    h, sth = _stage("s3", n)(yu, stu, gbu, yb, stb, gbb,
                             p['cW'][:HD], p['cW'][HD:], _pack1(p['cb']))
    xn = _stage("fin", n)(h, sth, _pack2(p['cg'], p['cbe']))
    return xn


def kernel(x0, x1, x2, up_index0, up_index1, b1_src, b1_dst, b2_src, b2_dst,
           batch0, batch1, batch2, params):
    prep_up0 = _prep_edges(up_index0[1], up_index0[0], N0)
    prep_up1 = _prep_edges(up_index1[1], up_index1[0], N1)
    prep_b1 = _prep_edges(b1_src, b1_dst, N1)
    prep_b2 = _prep_edges(b2_src, b2_dst, N2)

    for l in range(3):
        P = params['layers'][l]
        up0 = _msg(x0, prep_up0, N0)
        up1 = _msg(x1, prep_up1, N1)
        bm1 = _msg(x0, prep_b1, N1)
        bm2 = _msg(x1, prep_b2, N2)
        x0n = _conv_dim(x0, up0, None, P[0], N0)
        x1n = _conv_dim(x1, up1, bm1, P[1], N1)
        x2n = _conv_dim(x2, None, bm2, P[2], N2)
        x0, x1, x2 = x0n, x1n, x2n

    p0 = _pool(x0, _prep_pool(batch0, N0))
    p1 = _pool(x1, _prep_pool(batch1, N1))
    p2 = _pool(x2, _prep_pool(batch2, N2))

    (w10, b10), (w11, b11), (w12, b12) = params['lin1']
    w2, b2 = params['lin2']
    nc = w2.shape[1]
    w2p = jnp.zeros((2 * HD, HD), F32).at[:, :nc].set(w2)
    b2p = jnp.zeros((8, HD), F32).at[0, :nc].set(b2)
    out = _readout()(p0, p1, p2,
                     w10, _pack1(b10), w11, _pack1(b11), w12, _pack1(b12),
                     w2p, b2p)
    return out[:, :nc]


# speedup vs baseline: 1.5583x; 1.5583x over previous
import functools

import jax
import jax.numpy as jnp
from jax import lax
from jax.experimental import pallas as pl
from jax.experimental.pallas import tpu as pltpu
from jax.experimental.pallas import tpu_sc as plsc

F32 = jnp.float32
HD = 128
SB = 256
CH = 256
TN = 1000
SC_CHUNK = 256
SC_NW = 32
SC_ALIGN = SC_NW * SC_CHUNK
EPS = 1e-5

N0, N1, N2 = 10000, 160000, 50000
NB = 64


def _ceil_to(x, m):
    return (x + m - 1) // m * m


@functools.lru_cache(maxsize=None)
def _sc_gather(e_pad):
    b_per_w = e_pad // SC_NW
    n_chunks = b_per_w // SC_CHUNK
    mesh = plsc.VectorSubcoreMesh(core_axis_name="c", subcore_axis_name="s")

    @functools.partial(
        pl.kernel,
        mesh=mesh,
        out_type=jax.ShapeDtypeStruct((e_pad, HD), F32),
        scratch_types=[
            pltpu.VMEM((SC_CHUNK,), jnp.int32),
            pltpu.VMEM((SC_CHUNK,), jnp.int32),
            pltpu.VMEM((SC_CHUNK, HD), F32),
            pltpu.VMEM((SC_CHUNK, HD), F32),
            pltpu.SemaphoreType.DMA,
            pltpu.SemaphoreType.DMA,
        ],
    )
    def gather_kernel(table_hbm, idx_hbm, out_hbm, idx_v0, idx_v1, rows_v0,
                      rows_v1, sem0, sem1):
        wid = lax.axis_index("s") * 2 + lax.axis_index("c")
        base0 = wid * b_per_w
        idx_v = (idx_v0, idx_v1)
        rows_v = (rows_v0, rows_v1)
        sems = (sem0, sem1)
        prev = None
        for j in range(n_chunks):
            a = j % 2
            base = base0 + j * SC_CHUNK
            pltpu.sync_copy(idx_hbm.at[pl.ds(base, SC_CHUNK)], idx_v[a])
            h = pltpu.async_copy(table_hbm.at[idx_v[a]], rows_v[a], sems[a])
            if prev is not None:
                pj, ph = prev
                ph.wait()
                pltpu.sync_copy(
                    rows_v[pj % 2],
                    out_hbm.at[pl.ds(base0 + pj * SC_CHUNK, SC_CHUNK)])
            prev = (j, h)
        pj, ph = prev
        ph.wait()
        pltpu.sync_copy(rows_v[pj % 2],
                        out_hbm.at[pl.ds(base0 + pj * SC_CHUNK, SC_CHUNK)])

    return gather_kernel


def _segsum_body(gv_ref, cv_ref, fv_ref, ids_ref, rows_ref, out_ref, *, sb):
    v = pl.program_id(0)

    @pl.when(fv_ref[v] == 1)
    def _init():
        out_ref[...] = jnp.zeros_like(out_ref)

    row_iota = lax.broadcasted_iota(jnp.int32, (sb, CH), 0)
    local = ids_ref[0] - gv_ref[v] * sb
    oh = (row_iota == local).astype(F32)
    out_ref[...] += jnp.dot(oh, rows_ref[...], preferred_element_type=F32)


@functools.lru_cache(maxsize=None)
def _segsum(nb, sb, e_pad, interpret=False):
    v_pad = e_pad // CH + nb
    grid_spec = pltpu.PrefetchScalarGridSpec(
        num_scalar_prefetch=3,
        grid=(v_pad,),
        in_specs=[
            pl.BlockSpec((1, 1, CH), lambda v, gv, cv, fv: (cv[v], 0, 0)),
            pl.BlockSpec((CH, HD), lambda v, gv, cv, fv: (cv[v], 0)),
        ],
        out_specs=pl.BlockSpec((sb, HD), lambda v, gv, cv, fv: (gv[v], 0)),
    )
    return pl.pallas_call(
        functools.partial(_segsum_body, sb=sb),
        grid_spec=grid_spec,
        out_shape=jax.ShapeDtypeStruct(((nb + 1) * sb, HD), F32),
        interpret=interpret,
    )


def _visit_lists(starts, nb, e_pad):
    c0 = starts[:-1] // CH
    c1 = jnp.maximum((starts[1:] + CH - 1) // CH, c0 + 1)
    cnt = c1 - c0
    off = jnp.cumsum(cnt)
    v_pad = e_pad // CH + nb
    v = jnp.arange(v_pad, dtype=jnp.int32)
    g = jnp.searchsorted(off, v, side='right').astype(jnp.int32)
    total = off[nb - 1]
    real = v < total
    gc = jnp.minimum(g, nb - 1)
    prev_off = jnp.where(gc > 0, off[jnp.maximum(gc - 1, 0)], 0)
    c = c0[gc] + (v - prev_off)
    gv = jnp.where(real, g, nb).astype(jnp.int32)
    cv = jnp.where(real, c, 0).astype(jnp.int32)
    fv = ((v == prev_off) | ~real).astype(jnp.int32)
    return gv, cv, fv


def _affrelu(y, stats_ref, gb_ref, n_rows):
    inv_n = jnp.float32(1.0 / n_rows)
    mean = stats_ref[0:1, :] * inv_n
    var = stats_ref[1:2, :] * inv_n - mean * mean
    scale = gb_ref[0:1, :] * lax.rsqrt(var + EPS)
    shift = gb_ref[1:2, :] - mean * scale
    return jnp.maximum(y * scale + shift, 0.0)


def _acc_stats(y, st_ref):
    @pl.when(pl.program_id(0) == 0)
    def _init():
        st_ref[...] = jnp.zeros_like(st_ref)

    st_ref[0:1, :] += jnp.sum(y, axis=0, keepdims=True)
    st_ref[1:2, :] += jnp.sum(y * y, axis=0, keepdims=True)


def _s1_add_body(a_ref, b_ref, w_ref, bias_ref, y_ref, st_ref):
    x = a_ref[...] + b_ref[...]
    y = jnp.dot(x, w_ref[...], preferred_element_type=F32) + bias_ref[0:1, :]
    y_ref[...] = y
    _acc_stats(y, st_ref)


def _s1_single_body(a_ref, w_ref, bias_ref, y_ref, st_ref):
    y = (jnp.dot(a_ref[...], w_ref[...], preferred_element_type=F32)
         + bias_ref[0:1, :])
    y_ref[...] = y
    _acc_stats(y, st_ref)


def _s2_body(a_ref, sta_ref, gba_ref, w_ref, bias_ref, y_ref, st_ref, *,
             n_rows):
    x = _affrelu(a_ref[...], sta_ref, gba_ref, n_rows)
    y = jnp.dot(x, w_ref[...], preferred_element_type=F32) + bias_ref[0:1, :]
    y_ref[...] = y
    _acc_stats(y, st_ref)


def _s3_body(a_ref, sta_ref, gba_ref, b_ref, stb_ref, gbb_ref, w1_ref, w2_ref,
             bias_ref, y_ref, st_ref, *, n_rows):
    xa = _affrelu(a_ref[...], sta_ref, gba_ref, n_rows)
    xb = _affrelu(b_ref[...], stb_ref, gbb_ref, n_rows)
    y = (jnp.dot(xa, w1_ref[...], preferred_element_type=F32)
         + jnp.dot(xb, w2_ref[...], preferred_element_type=F32)
         + bias_ref[0:1, :])
    y_ref[...] = y
    _acc_stats(y, st_ref)


def _fin_body(a_ref, sta_ref, gba_ref, y_ref, *, n_rows):
    y_ref[...] = _affrelu(a_ref[...], sta_ref, gba_ref, n_rows)


def _row_spec():
    return pl.BlockSpec((TN, HD), lambda i: (i, 0))


def _full_spec(shape):
    return pl.BlockSpec(shape, lambda i: tuple(0 for _ in shape))


def _stats_out_specs():
    return [
        pl.BlockSpec((TN, HD), lambda i: (i, 0)),
        pl.BlockSpec((8, HD), lambda i: (0, 0)),
    ]


def _stats_out_shapes():
    return [
        None,
        jax.ShapeDtypeStruct((8, HD), F32),
    ]


@functools.lru_cache(maxsize=None)
def _stage(mode, n_rows, interpret=False):
    grid = (n_rows // TN,)
    y_shape = jax.ShapeDtypeStruct((n_rows, HD), F32)
    st_shape = jax.ShapeDtypeStruct((8, HD), F32)
    small = _full_spec((8, HD))
    wspec = _full_spec((HD, HD))
    if mode == "s1_add":
        body = _s1_add_body
        in_specs = [_row_spec(), _row_spec(), wspec, small]
    elif mode == "s1_single":
        body = _s1_single_body
        in_specs = [_row_spec(), wspec, small]
    elif mode == "s2":
        body = functools.partial(_s2_body, n_rows=n_rows)
        in_specs = [_row_spec(), small, small, wspec, small]
    elif mode == "s3":
        body = functools.partial(_s3_body, n_rows=n_rows)
        in_specs = [_row_spec(), small, small, _row_spec(), small, small,
                    wspec, wspec, small]
    elif mode == "fin":
        body = functools.partial(_fin_body, n_rows=n_rows)
        in_specs = [_row_spec(), small, small]
        return pl.pallas_call(
            body,
            grid=grid,
            in_specs=in_specs,
            out_specs=_row_spec(),
            out_shape=y_shape,
            interpret=interpret,
        )
    else:
        raise ValueError(mode)
    return pl.pallas_call(
        body,
        grid=grid,
        in_specs=in_specs,
        out_specs=[_row_spec(), pl.BlockSpec((8, HD), lambda i: (0, 0))],
        out_shape=[y_shape, st_shape],
        interpret=interpret,
    )


def _readout_body(p0, p1, p2, w10, b10, w11, b11, w12, b12, w2, b2, out_ref):
    h = jnp.maximum(jnp.dot(p0[...], w10[...],
                            preferred_element_type=F32) + b10[0:1, :], 0.0)
    h += jnp.maximum(jnp.dot(p1[...], w11[...],
                             preferred_element_type=F32) + b11[0:1, :], 0.0)
    h += jnp.maximum(jnp.dot(p2[...], w12[...],
                             preferred_element_type=F32) + b12[0:1, :], 0.0)
    out_ref[...] = (jnp.dot(h, w2[...], preferred_element_type=F32)
                    + b2[0:1, :])


@functools.lru_cache(maxsize=None)
def _readout(interpret=False):
    return pl.pallas_call(
        _readout_body,
        out_shape=jax.ShapeDtypeStruct((NB, HD), F32),
        interpret=interpret,
    )


def _pack2(top, bot):
    z = jnp.zeros((8, top.shape[0]), F32)
    return z.at[0].set(top).at[1].set(bot)


def _pack1(v):
    return jnp.zeros((8, v.shape[0]), F32).at[0].set(v)


def _prep_edges(src, dst, n_out):
    e = src.shape[0]
    nb = (n_out + SB - 1) // SB
    order = jnp.argsort(dst)
    src_s = jnp.take(src, order).astype(jnp.int32)
    dst_s = jnp.take(dst, order).astype(jnp.int32)
    e_pad = _ceil_to(e, SC_ALIGN)
    sent = jnp.int32(nb * SB)
    src_p = jnp.concatenate(
        [src_s, jnp.zeros((e_pad - e,), jnp.int32)])
    ids_p = jnp.concatenate(
        [dst_s, jnp.full((e_pad - e,), sent, jnp.int32)])
    ids3 = ids_p.reshape(e_pad // CH, 1, CH)
    bounds = jnp.arange(nb + 1, dtype=jnp.int32) * SB
    starts = jnp.searchsorted(dst_s, bounds).astype(jnp.int32)
    gv, cv, fv = _visit_lists(starts, nb, e_pad)
    return (src_p, ids3, gv, cv, fv, nb, e_pad)


def _msg(table, prep, n_out):
    src_p, ids3, gv, cv, fv, nb, e_pad = prep
    rows = _sc_gather(e_pad)(table, src_p)
    out = _segsum(nb, SB, e_pad)(gv, cv, fv, ids3, rows)
    return out[:n_out]


def _prep_pool(batch, n):
    e_pad = _ceil_to(n, CH)
    ids_p = jnp.concatenate(
        [batch.astype(jnp.int32),
         jnp.full((e_pad - n,), NB, jnp.int32)])
    ids3 = ids_p.reshape(e_pad // CH, 1, CH)
    starts = jnp.array([0, n], jnp.int32)
    gv, cv, fv = _visit_lists(starts, 1, e_pad)
    return ids3, gv, cv, fv, e_pad


def _pool(x, prep):
    ids3, gv, cv, fv, e_pad = prep
    xp = jnp.pad(x, ((0, e_pad - x.shape[0]), (0, 0)))
    return _segsum(1, NB, e_pad)(gv, cv, fv, ids3, xp)[:NB]


def _mlp_pair(x, msg, p, n):
    w1 = p['W1']
    if msg is None:
        y1, st1 = _stage("s1_single", n)(x, w1, _pack1(p['b1']))
    else:
        y1, st1 = _stage("s1_add", n)(x, msg, w1, _pack1(p['b1']))
    y2, st2 = _stage("s2", n)(y1, st1, _pack2(p['g1'], p['be1']), p['W2'],
                              _pack1(p['b2']))
    return y2, st2, _pack2(p['g2'], p['be2'])


def _conv_dim(x, up_msg, b_msg, p, n):
    yu, stu, gbu = _mlp_pair(x, up_msg, p['up'], n)
    yb, stb, gbb = _mlp_pair(x, b_msg, p['bnd'], n)
    h, sth = _stage("s3", n)(yu, stu, gbu, yb, stb, gbb,
                             p['cW'][:HD], p['cW'][HD:], _pack1(p['cb']))
    xn = _stage("fin", n)(h, sth, _pack2(p['cg'], p['cbe']))
    return xn


def kernel(x0, x1, x2, up_index0, up_index1, b1_src, b1_dst, b2_src, b2_dst,
           batch0, batch1, batch2, params):
    prep_up0 = _prep_edges(up_index0[1], up_index0[0], N0)
    prep_up1 = _prep_edges(up_index1[1], up_index1[0], N1)
    prep_b1 = _prep_edges(b1_src, b1_dst, N1)
    prep_b2 = _prep_edges(b2_src, b2_dst, N2)

    for l in range(3):
        P = params['layers'][l]
        up0 = _msg(x0, prep_up0, N0)
        up1 = _msg(x1, prep_up1, N1)
        bm1 = _msg(x0, prep_b1, N1)
        bm2 = _msg(x1, prep_b2, N2)
        x0n = _conv_dim(x0, up0, None, P[0], N0)
        x1n = _conv_dim(x1, up1, bm1, P[1], N1)
        x2n = _conv_dim(x2, None, bm2, P[2], N2)
        x0, x1, x2 = x0n, x1n, x2n

    p0 = _pool(x0, _prep_pool(batch0, N0))
    p1 = _pool(x1, _prep_pool(batch1, N1))
    p2 = _pool(x2, _prep_pool(batch2, N2))

    (w10, b10), (w11, b11), (w12, b12) = params['lin1']
    w2, b2 = params['lin2']
    nc = w2.shape[1]
    w2p = jnp.zeros((2 * HD, HD), F32).at[:, :nc].set(w2)
    b2p = jnp.zeros((8, HD), F32).at[0, :nc].set(b2)
    out = _readout()(p0, p1, p2,
                     w10, _pack1(b10), w11, _pack1(b11), w12, _pack1(b12),
                     w2p, b2p)
    return out[:, :nc]


# speedup vs baseline: 1.5800x; 1.0139x over previous
import functools

import jax
import jax.numpy as jnp
from jax import lax
from jax.experimental import pallas as pl
from jax.experimental.pallas import tpu as pltpu
from jax.experimental.pallas import tpu_sc as plsc

F32 = jnp.float32
HD = 128
SB = 256
CH = 256
TN = 1000
SC_CHUNK = 256
SC_NW = 32
SC_ALIGN = SC_NW * SC_CHUNK
EPS = 1e-5

N0, N1, N2 = 10000, 160000, 50000
NB = 64


def _ceil_to(x, m):
    return (x + m - 1) // m * m


@functools.lru_cache(maxsize=None)
def _sc_gather(e_pad):
    b_per_w = e_pad // SC_NW
    n_chunks = b_per_w // SC_CHUNK
    mesh = plsc.VectorSubcoreMesh(core_axis_name="c", subcore_axis_name="s")

    @functools.partial(
        pl.kernel,
        mesh=mesh,
        out_type=jax.ShapeDtypeStruct((e_pad, HD), F32),
        scratch_types=[
            pltpu.VMEM((SC_CHUNK,), jnp.int32),
            pltpu.VMEM((SC_CHUNK,), jnp.int32),
            pltpu.VMEM((SC_CHUNK, HD), F32),
            pltpu.VMEM((SC_CHUNK, HD), F32),
            pltpu.SemaphoreType.DMA,
            pltpu.SemaphoreType.DMA,
        ],
    )
    def gather_kernel(table_hbm, idx_hbm, out_hbm, idx_v0, idx_v1, rows_v0,
                      rows_v1, sem0, sem1):
        wid = lax.axis_index("s") * 2 + lax.axis_index("c")
        base0 = wid * b_per_w
        idx_v = (idx_v0, idx_v1)
        rows_v = (rows_v0, rows_v1)
        sems = (sem0, sem1)
        prev = None
        for j in range(n_chunks):
            a = j % 2
            base = base0 + j * SC_CHUNK
            pltpu.sync_copy(idx_hbm.at[pl.ds(base, SC_CHUNK)], idx_v[a])
            h = pltpu.async_copy(table_hbm.at[idx_v[a]], rows_v[a], sems[a])
            if prev is not None:
                pj, ph = prev
                ph.wait()
                pltpu.sync_copy(
                    rows_v[pj % 2],
                    out_hbm.at[pl.ds(base0 + pj * SC_CHUNK, SC_CHUNK)])
            prev = (j, h)
        pj, ph = prev
        ph.wait()
        pltpu.sync_copy(rows_v[pj % 2],
                        out_hbm.at[pl.ds(base0 + pj * SC_CHUNK, SC_CHUNK)])

    return gather_kernel


def _segsum_body(gv_ref, cv_ref, fv_ref, ids_ref, rows_ref, out_ref, *, sb):
    v = pl.program_id(0)

    @pl.when(fv_ref[v] == 1)
    def _init():
        out_ref[...] = jnp.zeros_like(out_ref)

    row_iota = lax.broadcasted_iota(jnp.int32, (sb, CH), 0)
    local = ids_ref[0] - gv_ref[v] * sb
    oh = (row_iota == local).astype(jnp.bfloat16)
    out_ref[...] += jnp.dot(oh, rows_ref[...].astype(jnp.bfloat16),
                            preferred_element_type=F32)


@functools.lru_cache(maxsize=None)
def _segsum(nb, sb, e_pad, interpret=False):
    v_pad = e_pad // CH + nb
    grid_spec = pltpu.PrefetchScalarGridSpec(
        num_scalar_prefetch=3,
        grid=(v_pad,),
        in_specs=[
            pl.BlockSpec((1, 1, CH), lambda v, gv, cv, fv: (cv[v], 0, 0)),
            pl.BlockSpec((CH, HD), lambda v, gv, cv, fv: (cv[v], 0)),
        ],
        out_specs=pl.BlockSpec((sb, HD), lambda v, gv, cv, fv: (gv[v], 0)),
    )
    return pl.pallas_call(
        functools.partial(_segsum_body, sb=sb),
        grid_spec=grid_spec,
        out_shape=jax.ShapeDtypeStruct(((nb + 1) * sb, HD), F32),
        interpret=interpret,
    )


def _visit_lists(starts, nb, e_pad):
    c0 = starts[:-1] // CH
    c1 = jnp.maximum((starts[1:] + CH - 1) // CH, c0 + 1)
    cnt = c1 - c0
    off = jnp.cumsum(cnt)
    v_pad = e_pad // CH + nb
    v = jnp.arange(v_pad, dtype=jnp.int32)
    g = jnp.searchsorted(off, v, side='right').astype(jnp.int32)
    total = off[nb - 1]
    real = v < total
    gc = jnp.minimum(g, nb - 1)
    prev_off = jnp.where(gc > 0, off[jnp.maximum(gc - 1, 0)], 0)
    c = c0[gc] + (v - prev_off)
    gv = jnp.where(real, g, nb).astype(jnp.int32)
    cv = jnp.where(real, c, 0).astype(jnp.int32)
    fv = ((v == prev_off) | ~real).astype(jnp.int32)
    return gv, cv, fv


def _affrelu(y, stats_ref, gb_ref, n_rows):
    inv_n = jnp.float32(1.0 / n_rows)
    mean = stats_ref[0:1, :] * inv_n
    var = stats_ref[1:2, :] * inv_n - mean * mean
    scale = gb_ref[0:1, :] * lax.rsqrt(var + EPS)
    shift = gb_ref[1:2, :] - mean * scale
    return jnp.maximum(y * scale + shift, 0.0)


def _acc_stats(y, st_ref):
    @pl.when(pl.program_id(0) == 0)
    def _init():
        st_ref[...] = jnp.zeros_like(st_ref)

    st_ref[0:1, :] += jnp.sum(y, axis=0, keepdims=True)
    st_ref[1:2, :] += jnp.sum(y * y, axis=0, keepdims=True)


def _s1_add_body(a_ref, b_ref, w_ref, bias_ref, y_ref, st_ref):
    x = a_ref[...] + b_ref[...]
    y = jnp.dot(x, w_ref[...], preferred_element_type=F32) + bias_ref[0:1, :]
    y_ref[...] = y
    _acc_stats(y, st_ref)


def _s1_single_body(a_ref, w_ref, bias_ref, y_ref, st_ref):
    y = (jnp.dot(a_ref[...], w_ref[...], preferred_element_type=F32)
         + bias_ref[0:1, :])
    y_ref[...] = y
    _acc_stats(y, st_ref)


def _s2_body(a_ref, sta_ref, gba_ref, w_ref, bias_ref, y_ref, st_ref, *,
             n_rows):
    x = _affrelu(a_ref[...], sta_ref, gba_ref, n_rows)
    y = jnp.dot(x, w_ref[...], preferred_element_type=F32) + bias_ref[0:1, :]
    y_ref[...] = y
    _acc_stats(y, st_ref)


def _s3_body(a_ref, sta_ref, gba_ref, b_ref, stb_ref, gbb_ref, w1_ref, w2_ref,
             bias_ref, y_ref, st_ref, *, n_rows):
    xa = _affrelu(a_ref[...], sta_ref, gba_ref, n_rows)
    xb = _affrelu(b_ref[...], stb_ref, gbb_ref, n_rows)
    y = (jnp.dot(xa, w1_ref[...], preferred_element_type=F32)
         + jnp.dot(xb, w2_ref[...], preferred_element_type=F32)
         + bias_ref[0:1, :])
    y_ref[...] = y
    _acc_stats(y, st_ref)


def _fin_body(a_ref, sta_ref, gba_ref, y_ref, *, n_rows):
    y_ref[...] = _affrelu(a_ref[...], sta_ref, gba_ref, n_rows)


def _row_spec():
    return pl.BlockSpec((TN, HD), lambda i: (i, 0))


def _full_spec(shape):
    return pl.BlockSpec(shape, lambda i: tuple(0 for _ in shape))


def _stats_out_specs():
    return [
        pl.BlockSpec((TN, HD), lambda i: (i, 0)),
        pl.BlockSpec((8, HD), lambda i: (0, 0)),
    ]


def _stats_out_shapes():
    return [
        None,
        jax.ShapeDtypeStruct((8, HD), F32),
    ]


@functools.lru_cache(maxsize=None)
def _stage(mode, n_rows, interpret=False):
    grid = (n_rows // TN,)
    y_shape = jax.ShapeDtypeStruct((n_rows, HD), F32)
    st_shape = jax.ShapeDtypeStruct((8, HD), F32)
    small = _full_spec((8, HD))
    wspec = _full_spec((HD, HD))
    if mode == "s1_add":
        body = _s1_add_body
        in_specs = [_row_spec(), _row_spec(), wspec, small]
    elif mode == "s1_single":
        body = _s1_single_body
        in_specs = [_row_spec(), wspec, small]
    elif mode == "s2":
        body = functools.partial(_s2_body, n_rows=n_rows)
        in_specs = [_row_spec(), small, small, wspec, small]
    elif mode == "s3":
        body = functools.partial(_s3_body, n_rows=n_rows)
        in_specs = [_row_spec(), small, small, _row_spec(), small, small,
                    wspec, wspec, small]
    elif mode == "fin":
        body = functools.partial(_fin_body, n_rows=n_rows)
        in_specs = [_row_spec(), small, small]
        return pl.pallas_call(
            body,
            grid=grid,
            in_specs=in_specs,
            out_specs=_row_spec(),
            out_shape=y_shape,
            interpret=interpret,
        )
    else:
        raise ValueError(mode)
    return pl.pallas_call(
        body,
        grid=grid,
        in_specs=in_specs,
        out_specs=[_row_spec(), pl.BlockSpec((8, HD), lambda i: (0, 0))],
        out_shape=[y_shape, st_shape],
        interpret=interpret,
    )


def _readout_body(p0, p1, p2, w10, b10, w11, b11, w12, b12, w2, b2, out_ref):
    h = jnp.maximum(jnp.dot(p0[...], w10[...],
                            preferred_element_type=F32) + b10[0:1, :], 0.0)
    h += jnp.maximum(jnp.dot(p1[...], w11[...],
                             preferred_element_type=F32) + b11[0:1, :], 0.0)
    h += jnp.maximum(jnp.dot(p2[...], w12[...],
                             preferred_element_type=F32) + b12[0:1, :], 0.0)
    out_ref[...] = (jnp.dot(h, w2[...], preferred_element_type=F32)
                    + b2[0:1, :])


@functools.lru_cache(maxsize=None)
def _readout(interpret=False):
    return pl.pallas_call(
        _readout_body,
        out_shape=jax.ShapeDtypeStruct((NB, HD), F32),
        interpret=interpret,
    )


def _pack2(top, bot):
    z = jnp.zeros((8, top.shape[0]), F32)
    return z.at[0].set(top).at[1].set(bot)


def _pack1(v):
    return jnp.zeros((8, v.shape[0]), F32).at[0].set(v)


def _prep_edges(src, dst, n_out):
    e = src.shape[0]
    nb = (n_out + SB - 1) // SB
    dst_s, src_s = lax.sort((dst.astype(jnp.int32), src.astype(jnp.int32)),
                            num_keys=1)
    e_pad = _ceil_to(e, SC_ALIGN)
    sent = jnp.int32(nb * SB)
    src_p = jnp.concatenate(
        [src_s, jnp.zeros((e_pad - e,), jnp.int32)])
    ids_p = jnp.concatenate(
        [dst_s, jnp.full((e_pad - e,), sent, jnp.int32)])
    ids3 = ids_p.reshape(e_pad // CH, 1, CH)
    bounds = jnp.arange(nb + 1, dtype=jnp.int32) * SB
    starts = jnp.searchsorted(dst_s, bounds).astype(jnp.int32)
    gv, cv, fv = _visit_lists(starts, nb, e_pad)
    return (src_p, ids3, gv, cv, fv, nb, e_pad)


def _msg(table, prep, n_out):
    src_p, ids3, gv, cv, fv, nb, e_pad = prep
    rows = _sc_gather(e_pad)(table, src_p)
    out = _segsum(nb, SB, e_pad)(gv, cv, fv, ids3, rows)
    return out[:n_out]


def _prep_pool(batch, n):
    e_pad = _ceil_to(n, CH)
    ids_p = jnp.concatenate(
        [batch.astype(jnp.int32),
         jnp.full((e_pad - n,), NB, jnp.int32)])
    ids3 = ids_p.reshape(e_pad // CH, 1, CH)
    starts = jnp.array([0, n], jnp.int32)
    gv, cv, fv = _visit_lists(starts, 1, e_pad)
    return ids3, gv, cv, fv, e_pad


def _pool(x, prep):
    ids3, gv, cv, fv, e_pad = prep
    xp = jnp.pad(x, ((0, e_pad - x.shape[0]), (0, 0)))
    return _segsum(1, NB, e_pad)(gv, cv, fv, ids3, xp)[:NB]


def _mlp_pair(x, msg, p, n):
    w1 = p['W1']
    if msg is None:
        y1, st1 = _stage("s1_single", n)(x, w1, _pack1(p['b1']))
    else:
        y1, st1 = _stage("s1_add", n)(x, msg, w1, _pack1(p['b1']))
    y2, st2 = _stage("s2", n)(y1, st1, _pack2(p['g1'], p['be1']), p['W2'],
                              _pack1(p['b2']))
    return y2, st2, _pack2(p['g2'], p['be2'])


def _conv_dim(x, up_msg, b_msg, p, n):
    yu, stu, gbu = _mlp_pair(x, up_msg, p['up'], n)
    yb, stb, gbb = _mlp_pair(x, b_msg, p['bnd'], n)
    h, sth = _stage("s3", n)(yu, stu, gbu, yb, stb, gbb,
                             p['cW'][:HD], p['cW'][HD:], _pack1(p['cb']))
    xn = _stage("fin", n)(h, sth, _pack2(p['cg'], p['cbe']))
    return xn


def kernel(x0, x1, x2, up_index0, up_index1, b1_src, b1_dst, b2_src, b2_dst,
           batch0, batch1, batch2, params):
    prep_up0 = _prep_edges(up_index0[1], up_index0[0], N0)
    prep_up1 = _prep_edges(up_index1[1], up_index1[0], N1)
    prep_b1 = _prep_edges(b1_src, b1_dst, N1)
    prep_b2 = _prep_edges(b2_src, b2_dst, N2)

    for l in range(3):
        P = params['layers'][l]
        up0 = _msg(x0, prep_up0, N0)
        up1 = _msg(x1, prep_up1, N1)
        bm1 = _msg(x0, prep_b1, N1)
        bm2 = _msg(x1, prep_b2, N2)
        x0n = _conv_dim(x0, up0, None, P[0], N0)
        x1n = _conv_dim(x1, up1, bm1, P[1], N1)
        x2n = _conv_dim(x2, None, bm2, P[2], N2)
        x0, x1, x2 = x0n, x1n, x2n

    p0 = _pool(x0, _prep_pool(batch0, N0))
    p1 = _pool(x1, _prep_pool(batch1, N1))
    p2 = _pool(x2, _prep_pool(batch2, N2))

    (w10, b10), (w11, b11), (w12, b12) = params['lin1']
    w2, b2 = params['lin2']
    nc = w2.shape[1]
    w2p = jnp.zeros((2 * HD, HD), F32).at[:, :nc].set(w2)
    b2p = jnp.zeros((8, HD), F32).at[0, :nc].set(b2)
    out = _readout()(p0, p1, p2,
                     w10, _pack1(b10), w11, _pack1(b11), w12, _pack1(b12),
                     w2p, b2p)
    return out[:, :nc]


# speedup vs baseline: 1.9468x; 1.2322x over previous
import functools

import jax
import jax.numpy as jnp
from jax import lax
from jax.experimental import pallas as pl
from jax.experimental.pallas import tpu as pltpu
from jax.experimental.pallas import tpu_sc as plsc

F32 = jnp.float32
HD = 128
SB = 256
CH = 512
TN = 1000
SC_CHUNK = 256
SC_NW = 32
SC_ALIGN = SC_NW * SC_CHUNK
EPS = 1e-5

N0, N1, N2 = 10000, 160000, 50000
NB = 64


def _ceil_to(x, m):
    return (x + m - 1) // m * m


@functools.lru_cache(maxsize=None)
def _sc_gather(e_pad):
    b_per_w = e_pad // SC_NW
    n_chunks = b_per_w // SC_CHUNK
    mesh = plsc.VectorSubcoreMesh(core_axis_name="c", subcore_axis_name="s")

    @functools.partial(
        pl.kernel,
        mesh=mesh,
        out_type=jax.ShapeDtypeStruct((e_pad, HD), F32),
        scratch_types=[
            pltpu.VMEM((SC_CHUNK,), jnp.int32),
            pltpu.VMEM((SC_CHUNK,), jnp.int32),
            pltpu.VMEM((SC_CHUNK, HD), F32),
            pltpu.VMEM((SC_CHUNK, HD), F32),
            pltpu.SemaphoreType.DMA,
            pltpu.SemaphoreType.DMA,
        ],
    )
    def gather_kernel(table_hbm, idx_hbm, out_hbm, idx_v0, idx_v1, rows_v0,
                      rows_v1, sem0, sem1):
        wid = lax.axis_index("s") * 2 + lax.axis_index("c")
        base0 = wid * b_per_w
        idx_v = (idx_v0, idx_v1)
        rows_v = (rows_v0, rows_v1)
        sems = (sem0, sem1)
        prev = None
        for j in range(n_chunks):
            a = j % 2
            base = base0 + j * SC_CHUNK
            pltpu.sync_copy(idx_hbm.at[pl.ds(base, SC_CHUNK)], idx_v[a])
            h = pltpu.async_copy(table_hbm.at[idx_v[a]], rows_v[a], sems[a])
            if prev is not None:
                pj, ph = prev
                ph.wait()
                pltpu.sync_copy(
                    rows_v[pj % 2],
                    out_hbm.at[pl.ds(base0 + pj * SC_CHUNK, SC_CHUNK)])
            prev = (j, h)
        pj, ph = prev
        ph.wait()
        pltpu.sync_copy(rows_v[pj % 2],
                        out_hbm.at[pl.ds(base0 + pj * SC_CHUNK, SC_CHUNK)])

    return gather_kernel


def _segsum_body(gv_ref, cv_ref, fv_ref, ids_ref, rows_ref, out_ref, *, sb):
    v = pl.program_id(0)

    @pl.when(fv_ref[v] == 1)
    def _init():
        out_ref[...] = jnp.zeros_like(out_ref)

    row_iota = lax.broadcasted_iota(jnp.int32, (sb, CH), 0)
    local = ids_ref[0] - gv_ref[v] * sb
    oh = (row_iota == local).astype(jnp.bfloat16)
    out_ref[...] += jnp.dot(oh, rows_ref[...].astype(jnp.bfloat16),
                            preferred_element_type=F32)


@functools.lru_cache(maxsize=None)
def _segsum(nb, sb, e_pad, interpret=False):
    v_pad = e_pad // CH + nb
    grid_spec = pltpu.PrefetchScalarGridSpec(
        num_scalar_prefetch=3,
        grid=(v_pad,),
        in_specs=[
            pl.BlockSpec((1, 1, CH), lambda v, gv, cv, fv: (cv[v], 0, 0)),
            pl.BlockSpec((CH, HD), lambda v, gv, cv, fv: (cv[v], 0)),
        ],
        out_specs=pl.BlockSpec((sb, HD), lambda v, gv, cv, fv: (gv[v], 0)),
    )
    return pl.pallas_call(
        functools.partial(_segsum_body, sb=sb),
        grid_spec=grid_spec,
        out_shape=jax.ShapeDtypeStruct(((nb + 1) * sb, HD), F32),
        interpret=interpret,
    )


def _visit_lists(starts, nb, e_pad):
    c0 = starts[:-1] // CH
    c1 = jnp.maximum((starts[1:] + CH - 1) // CH, c0 + 1)
    cnt = c1 - c0
    off = jnp.cumsum(cnt)
    v_pad = e_pad // CH + nb
    v = jnp.arange(v_pad, dtype=jnp.int32)
    g = jnp.searchsorted(off, v, side='right').astype(jnp.int32)
    total = off[nb - 1]
    real = v < total
    gc = jnp.minimum(g, nb - 1)
    prev_off = jnp.where(gc > 0, off[jnp.maximum(gc - 1, 0)], 0)
    c = c0[gc] + (v - prev_off)
    gv = jnp.where(real, g, nb).astype(jnp.int32)
    cv = jnp.where(real, c, 0).astype(jnp.int32)
    fv = ((v == prev_off) | ~real).astype(jnp.int32)
    return gv, cv, fv


def _affrelu(y, stats_ref, gb_ref, n_rows):
    inv_n = jnp.float32(1.0 / n_rows)
    mean = stats_ref[0:1, :] * inv_n
    var = stats_ref[1:2, :] * inv_n - mean * mean
    scale = gb_ref[0:1, :] * lax.rsqrt(var + EPS)
    shift = gb_ref[1:2, :] - mean * scale
    return jnp.maximum(y * scale + shift, 0.0)


def _acc_stats(y, st_ref):
    @pl.when(pl.program_id(0) == 0)
    def _init():
        st_ref[...] = jnp.zeros_like(st_ref)

    st_ref[0:1, :] += jnp.sum(y, axis=0, keepdims=True)
    st_ref[1:2, :] += jnp.sum(y * y, axis=0, keepdims=True)


def _s1_add_body(a_ref, b_ref, w_ref, bias_ref, y_ref, st_ref):
    x = a_ref[...] + b_ref[...]
    y = jnp.dot(x, w_ref[...], preferred_element_type=F32) + bias_ref[0:1, :]
    y_ref[...] = y
    _acc_stats(y, st_ref)


def _s1_single_body(a_ref, w_ref, bias_ref, y_ref, st_ref):
    y = (jnp.dot(a_ref[...], w_ref[...], preferred_element_type=F32)
         + bias_ref[0:1, :])
    y_ref[...] = y
    _acc_stats(y, st_ref)


def _s2_body(a_ref, sta_ref, gba_ref, w_ref, bias_ref, y_ref, st_ref, *,
             n_rows):
    x = _affrelu(a_ref[...], sta_ref, gba_ref, n_rows)
    y = jnp.dot(x, w_ref[...], preferred_element_type=F32) + bias_ref[0:1, :]
    y_ref[...] = y
    _acc_stats(y, st_ref)


def _s3_body(a_ref, sta_ref, gba_ref, b_ref, stb_ref, gbb_ref, w1_ref, w2_ref,
             bias_ref, y_ref, st_ref, *, n_rows):
    xa = _affrelu(a_ref[...], sta_ref, gba_ref, n_rows)
    xb = _affrelu(b_ref[...], stb_ref, gbb_ref, n_rows)
    y = (jnp.dot(xa, w1_ref[...], preferred_element_type=F32)
         + jnp.dot(xb, w2_ref[...], preferred_element_type=F32)
         + bias_ref[0:1, :])
    y_ref[...] = y
    _acc_stats(y, st_ref)


def _fin_body(a_ref, sta_ref, gba_ref, y_ref, *, n_rows):
    y_ref[...] = _affrelu(a_ref[...], sta_ref, gba_ref, n_rows)


def _row_spec():
    return pl.BlockSpec((TN, HD), lambda i: (i, 0))


def _full_spec(shape):
    return pl.BlockSpec(shape, lambda i: tuple(0 for _ in shape))


def _stats_out_specs():
    return [
        pl.BlockSpec((TN, HD), lambda i: (i, 0)),
        pl.BlockSpec((8, HD), lambda i: (0, 0)),
    ]


def _stats_out_shapes():
    return [
        None,
        jax.ShapeDtypeStruct((8, HD), F32),
    ]


@functools.lru_cache(maxsize=None)
def _stage(mode, n_rows, interpret=False):
    grid = (n_rows // TN,)
    y_shape = jax.ShapeDtypeStruct((n_rows, HD), F32)
    st_shape = jax.ShapeDtypeStruct((8, HD), F32)
    small = _full_spec((8, HD))
    wspec = _full_spec((HD, HD))
    if mode == "s1_add":
        body = _s1_add_body
        in_specs = [_row_spec(), _row_spec(), wspec, small]
    elif mode == "s1_single":
        body = _s1_single_body
        in_specs = [_row_spec(), wspec, small]
    elif mode == "s2":
        body = functools.partial(_s2_body, n_rows=n_rows)
        in_specs = [_row_spec(), small, small, wspec, small]
    elif mode == "s3":
        body = functools.partial(_s3_body, n_rows=n_rows)
        in_specs = [_row_spec(), small, small, _row_spec(), small, small,
                    wspec, wspec, small]
    elif mode == "fin":
        body = functools.partial(_fin_body, n_rows=n_rows)
        in_specs = [_row_spec(), small, small]
        return pl.pallas_call(
            body,
            grid=grid,
            in_specs=in_specs,
            out_specs=_row_spec(),
            out_shape=y_shape,
            interpret=interpret,
        )
    else:
        raise ValueError(mode)
    return pl.pallas_call(
        body,
        grid=grid,
        in_specs=in_specs,
        out_specs=[_row_spec(), pl.BlockSpec((8, HD), lambda i: (0, 0))],
        out_shape=[y_shape, st_shape],
        interpret=interpret,
    )


def _readout_body(p0, p1, p2, w10, b10, w11, b11, w12, b12, w2, b2, out_ref):
    h = jnp.maximum(jnp.dot(p0[...], w10[...],
                            preferred_element_type=F32) + b10[0:1, :], 0.0)
    h += jnp.maximum(jnp.dot(p1[...], w11[...],
                             preferred_element_type=F32) + b11[0:1, :], 0.0)
    h += jnp.maximum(jnp.dot(p2[...], w12[...],
                             preferred_element_type=F32) + b12[0:1, :], 0.0)
    out_ref[...] = (jnp.dot(h, w2[...], preferred_element_type=F32)
                    + b2[0:1, :])


@functools.lru_cache(maxsize=None)
def _readout(interpret=False):
    return pl.pallas_call(
        _readout_body,
        out_shape=jax.ShapeDtypeStruct((NB, HD), F32),
        interpret=interpret,
    )


def _pack2(top, bot):
    z = jnp.zeros((8, top.shape[0]), F32)
    return z.at[0].set(top).at[1].set(bot)


def _pack1(v):
    return jnp.zeros((8, v.shape[0]), F32).at[0].set(v)


def _prep_edges(src, dst, n_out):
    e = src.shape[0]
    nb = (n_out + SB - 1) // SB
    dst_s, src_s = lax.sort((dst.astype(jnp.int32), src.astype(jnp.int32)),
                            num_keys=1)
    e_pad = _ceil_to(e, SC_ALIGN)
    sent = jnp.int32(nb * SB)
    src_p = jnp.concatenate(
        [src_s, jnp.zeros((e_pad - e,), jnp.int32)])
    ids_p = jnp.concatenate(
        [dst_s, jnp.full((e_pad - e,), sent, jnp.int32)])
    ids3 = ids_p.reshape(e_pad // CH, 1, CH)
    bounds = jnp.arange(nb + 1, dtype=jnp.int32) * SB
    starts = jnp.searchsorted(dst_s, bounds).astype(jnp.int32)
    gv, cv, fv = _visit_lists(starts, nb, e_pad)
    return (src_p, ids3, gv, cv, fv, nb, e_pad)


def _msg(table, prep, n_out):
    src_p, ids3, gv, cv, fv, nb, e_pad = prep
    rows = _sc_gather(e_pad)(table, src_p)
    out = _segsum(nb, SB, e_pad)(gv, cv, fv, ids3, rows)
    return out[:n_out]


def _prep_pool(batch, n):
    e_pad = _ceil_to(n, CH)
    ids_p = jnp.concatenate(
        [batch.astype(jnp.int32),
         jnp.full((e_pad - n,), NB, jnp.int32)])
    ids3 = ids_p.reshape(e_pad // CH, 1, CH)
    starts = jnp.array([0, n], jnp.int32)
    gv, cv, fv = _visit_lists(starts, 1, e_pad)
    return ids3, gv, cv, fv, e_pad


def _pool(x, prep):
    ids3, gv, cv, fv, e_pad = prep
    xp = jnp.pad(x, ((0, e_pad - x.shape[0]), (0, 0)))
    return _segsum(1, NB, e_pad)(gv, cv, fv, ids3, xp)[:NB]


def _mlp_pair(x, msg, p, n):
    w1 = p['W1']
    if msg is None:
        y1, st1 = _stage("s1_single", n)(x, w1, _pack1(p['b1']))
    else:
        y1, st1 = _stage("s1_add", n)(x, msg, w1, _pack1(p['b1']))
    y2, st2 = _stage("s2", n)(y1, st1, _pack2(p['g1'], p['be1']), p['W2'],
                              _pack1(p['b2']))
    return y2, st2, _pack2(p['g2'], p['be2'])


def _conv_dim(x, up_msg, b_msg, p, n):
    yu, stu, gbu = _mlp_pair(x, up_msg, p['up'], n)
    yb, stb, gbb = _mlp_pair(x, b_msg, p['bnd'], n)
    h, sth = _stage("s3", n)(yu, stu, gbu, yb, stb, gbb,
                             p['cW'][:HD], p['cW'][HD:], _pack1(p['cb']))
    xn = _stage("fin", n)(h, sth, _pack2(p['cg'], p['cbe']))
    return xn


def kernel(x0, x1, x2, up_index0, up_index1, b1_src, b1_dst, b2_src, b2_dst,
           batch0, batch1, batch2, params):
    prep_up0 = _prep_edges(up_index0[1], up_index0[0], N0)
    prep_up1 = _prep_edges(up_index1[1], up_index1[0], N1)
    prep_b1 = _prep_edges(b1_src, b1_dst, N1)
    prep_b2 = _prep_edges(b2_src, b2_dst, N2)

    for l in range(3):
        P = params['layers'][l]
        up0 = _msg(x0, prep_up0, N0)
        up1 = _msg(x1, prep_up1, N1)
        bm1 = _msg(x0, prep_b1, N1)
        bm2 = _msg(x1, prep_b2, N2)
        x0n = _conv_dim(x0, up0, None, P[0], N0)
        x1n = _conv_dim(x1, up1, bm1, P[1], N1)
        x2n = _conv_dim(x2, None, bm2, P[2], N2)
        x0, x1, x2 = x0n, x1n, x2n

    p0 = _pool(x0, _prep_pool(batch0, N0))
    p1 = _pool(x1, _prep_pool(batch1, N1))
    p2 = _pool(x2, _prep_pool(batch2, N2))

    (w10, b10), (w11, b11), (w12, b12) = params['lin1']
    w2, b2 = params['lin2']
    nc = w2.shape[1]
    w2p = jnp.zeros((2 * HD, HD), F32).at[:, :nc].set(w2)
    b2p = jnp.zeros((8, HD), F32).at[0, :nc].set(b2)
    out = _readout()(p0, p1, p2,
                     w10, _pack1(b10), w11, _pack1(b11), w12, _pack1(b12),
                     w2p, b2p)
    return out[:, :nc]
